# baseline (speedup 1.0000x reference)
import dataclasses
import functools

import jax
import jax.numpy as jnp
from jax import lax
from jax.experimental import pallas as pl
from jax.experimental.pallas import tpu as pltpu
from jax.experimental.pallas import tpu_sc as plsc

B = 32
S = 2048
VOCAB = 32000
EMBED = 1024
RH = 512
NM = 12
TOP_K = 3

VB = 1280
NPOOL = VOCAB // VB
NCORE = 2
DHALF = EMBED // NCORE
MHALF = NM // NCORE

_LANES = 16


def _histogram(input_ids):
    ids3 = input_ids.reshape(B, S // _LANES, _LANES)
    mesh = plsc.VectorSubcoreMesh(core_axis_name="c", subcore_axis_name="s")
    cp = pltpu.CompilerParams()
    if "needs_layout_passes" in pltpu.CompilerParams.__dataclass_fields__:
        cp = dataclasses.replace(cp, needs_layout_passes=False)

    @functools.partial(
        pl.kernel,
        out_type=jax.ShapeDtypeStruct((B, VOCAB), jnp.float32),
        mesh=mesh,
        compiler_params=cp,
        scratch_types=[
            pltpu.VMEM((S // _LANES, _LANES), jnp.int32),
            pltpu.VMEM((VOCAB,), jnp.float32),
            pltpu.SemaphoreType.DMA,
        ],
    )
    def hist(ids_hbm, counts_hbm, ids_v, counts_v, sem):
        cid = lax.axis_index("c")
        sid = lax.axis_index("s")
        row = cid * 16 + sid
        cp_ids = pltpu.async_copy(ids_hbm.at[row], ids_v, sem)

        zeros = jnp.zeros((_LANES,), jnp.float32)
        _ZU = 8

        @pl.loop(0, VOCAB // (_ZU * _LANES))
        def _(i):
            for u in range(_ZU):
                counts_v[pl.ds(i * (_ZU * _LANES) + u * _LANES, _LANES)] = zeros

        cp_ids.wait()
        ones = jnp.ones((_LANES,), jnp.float32)
        _SU = 4

        @pl.loop(0, S // (_SU * _LANES))
        def _(j):
            for u in range(_SU):
                plsc.addupdate_scatter(counts_v, [ids_v[j * _SU + u, :]], ones)

        pltpu.sync_copy(counts_v, counts_hbm.at[row])

    return hist(ids3)


def _dd(a, b):
    return lax.dot_general(
        a, b, (((1,), (0,)), ((), ())),
        preferred_element_type=jnp.float32,
        precision=lax.Precision.DEFAULT)


def _pool_body(counts_ref, emb_ref, pooled_ref, acc):
    t = pl.program_id(1)

    @pl.when(t == 0)
    def _():
        acc[...] = jnp.zeros_like(acc)

    c = counts_ref[...]
    e = emb_ref[...]
    cB = jnp.floor(c * (1.0 / 256.0)) * 256.0
    cA = c - cB
    e1 = e.astype(jnp.bfloat16).astype(jnp.float32)
    r1 = e - e1
    e2 = r1.astype(jnp.bfloat16).astype(jnp.float32)
    r2 = r1 - e2
    acc[...] += _dd(cA, e1) + _dd(cA, e2) + _dd(cA, r2)

    @pl.when(jnp.max(c) > 255.5)
    def _():
        acc[...] += _dd(cB, e1) + _dd(cB, e2) + _dd(cB, r2)

    @pl.when(t == NPOOL - 1)
    def _():
        pooled_ref[...] = acc[...] * (1.0 / S)


def _route_body(pooled_ref, wr1_ref, br1_ref, wp_ref, bp_ref,
                w1_ref, b1_ref, w2_ref, b2_ref,
                final_ref, logits_ref, act_ref,
                facc, act_s):
    t = pl.program_id(1)

    @pl.when(t == 0)
    def _():
        x = pooled_ref[...]
        xb = x.astype(jnp.bfloat16)
        h = jnp.maximum(
            _dd(xb, wr1_ref[...].astype(jnp.bfloat16)) + br1_ref[...], 0.0)
        lg = _dd(h.astype(jnp.bfloat16),
                 wp_ref[...].astype(jnp.bfloat16)) + bp_ref[...]
        logits_ref[0] = lg
        col = lax.broadcasted_iota(jnp.int32, lg.shape, 1)
        m = lg
        a = jnp.zeros_like(lg)
        for _ in range(TOP_K):
            mx = jnp.max(m, axis=1, keepdims=True)
            first = jnp.min(jnp.where(m == mx, col, NM), axis=1, keepdims=True)
            sel = col == first
            a = jnp.where(sel, 1.0, a)
            m = jnp.where(sel, -jnp.inf, m)
        act_ref[0] = a
        act_s[...] = a

    k = pl.program_id(0)
    e_idx = k * MHALF + t
    xb = pooled_ref[...].astype(jnp.bfloat16)
    h1 = jnp.maximum(
        _dd(xb, w1_ref[0].astype(jnp.bfloat16)) + b1_ref[0], 0.0)
    o = _dd(h1.astype(jnp.bfloat16), w2_ref[0].astype(jnp.bfloat16)) + b2_ref[0]
    col = lax.broadcasted_iota(jnp.int32, act_s.shape, 1)
    w = jnp.sum(jnp.where(col == e_idx, act_s[...], 0.0),
                axis=1, keepdims=True)

    @pl.when(t == 0)
    def _():
        facc[...] = w * o

    @pl.when(t > 0)
    def _():
        facc[...] += w * o

    @pl.when(t == MHALF - 1)
    def _():
        final_ref[0] = facc[...]


def kernel(input_ids, working_memory, affective_context, emb_table,
           Wr1, br1, Wp, bp, Wsg, W1, b1, W2, b2):
    counts = _histogram(input_ids)

    pooled = pl.pallas_call(
        _pool_body,
        grid=(NCORE, NPOOL),
        in_specs=[
            pl.BlockSpec((B, VB), lambda k, t: (0, t)),
            pl.BlockSpec((VB, DHALF), lambda k, t: (t, k)),
        ],
        out_specs=pl.BlockSpec((B, DHALF), lambda k, t: (0, k)),
        out_shape=jax.ShapeDtypeStruct((B, EMBED), jnp.float32),
        scratch_shapes=[pltpu.VMEM((B, DHALF), jnp.float32)],
        compiler_params=pltpu.CompilerParams(
            dimension_semantics=("parallel", "arbitrary")),
    )(counts, emb_table)

    final2, logits2, act2 = pl.pallas_call(
        _route_body,
        grid=(NCORE, MHALF),
        in_specs=[
            pl.BlockSpec((B, EMBED), lambda k, t: (0, 0)),
            pl.BlockSpec((EMBED, RH), lambda k, t: (0, 0)),
            pl.BlockSpec((1, RH), lambda k, t: (0, 0)),
            pl.BlockSpec((RH, NM), lambda k, t: (0, 0)),
            pl.BlockSpec((1, NM), lambda k, t: (0, 0)),
            pl.BlockSpec((1, EMBED, EMBED), lambda k, t: (k * MHALF + t, 0, 0)),
            pl.BlockSpec((1, 1, EMBED), lambda k, t: (k * MHALF + t, 0, 0)),
            pl.BlockSpec((1, EMBED, EMBED), lambda k, t: (k * MHALF + t, 0, 0)),
            pl.BlockSpec((1, 1, EMBED), lambda k, t: (k * MHALF + t, 0, 0)),
        ],
        out_specs=[
            pl.BlockSpec((1, B, EMBED), lambda k, t: (k, 0, 0)),
            pl.BlockSpec((1, B, NM), lambda k, t: (k, 0, 0)),
            pl.BlockSpec((1, B, NM), lambda k, t: (k, 0, 0)),
        ],
        out_shape=[
            jax.ShapeDtypeStruct((NCORE, B, EMBED), jnp.float32),
            jax.ShapeDtypeStruct((NCORE, B, NM), jnp.float32),
            jax.ShapeDtypeStruct((NCORE, B, NM), jnp.float32),
        ],
        scratch_shapes=[
            pltpu.VMEM((B, EMBED), jnp.float32),
            pltpu.VMEM((B, NM), jnp.float32),
        ],
        compiler_params=pltpu.CompilerParams(
            dimension_semantics=("parallel", "arbitrary")),
    )(pooled, Wr1, br1.reshape(1, RH), Wp, bp.reshape(1, NM),
      W1, b1.reshape(NM, 1, EMBED), W2, b2.reshape(NM, 1, EMBED))

    final = final2[0] + final2[1]
    return final, logits2[0], act2[0]


# speedup vs baseline: 1.1382x; 1.1382x over previous
import dataclasses
import functools

import jax
import jax.numpy as jnp
from jax import lax
from jax.experimental import pallas as pl
from jax.experimental.pallas import tpu as pltpu
from jax.experimental.pallas import tpu_sc as plsc

B = 32
S = 2048
VOCAB = 32000
EMBED = 1024
RH = 512
NM = 12
TOP_K = 3

VB = 1280
NPOOL = VOCAB // VB
GRID = NPOOL + NM

_LANES = 16


def _histogram(input_ids):
    ids3 = input_ids.reshape(B, S // _LANES, _LANES)
    mesh = plsc.VectorSubcoreMesh(core_axis_name="c", subcore_axis_name="s")
    cp = pltpu.CompilerParams()
    if "needs_layout_passes" in pltpu.CompilerParams.__dataclass_fields__:
        cp = dataclasses.replace(cp, needs_layout_passes=False)

    @functools.partial(
        pl.kernel,
        out_type=jax.ShapeDtypeStruct((B, VOCAB), jnp.float32),
        mesh=mesh,
        compiler_params=cp,
        scratch_types=[
            pltpu.VMEM((S // _LANES, _LANES), jnp.int32),
            pltpu.VMEM((VOCAB,), jnp.float32),
            pltpu.SemaphoreType.DMA,
        ],
    )
    def hist(ids_hbm, counts_hbm, ids_v, counts_v, sem):
        cid = lax.axis_index("c")
        sid = lax.axis_index("s")
        row = cid * 16 + sid
        cp = pltpu.async_copy(ids_hbm.at[row], ids_v, sem)

        zeros = jnp.zeros((_LANES,), jnp.float32)
        _ZU = 8

        @pl.loop(0, VOCAB // (_ZU * _LANES))
        def _(i):
            for u in range(_ZU):
                counts_v[pl.ds(i * (_ZU * _LANES) + u * _LANES, _LANES)] = zeros

        cp.wait()
        ones = jnp.ones((_LANES,), jnp.float32)
        _SU = 4

        @pl.loop(0, S // (_SU * _LANES))
        def _(j):
            for u in range(_SU):
                plsc.addupdate_scatter(counts_v, [ids_v[j * _SU + u, :]], ones)

        pltpu.sync_copy(counts_v, counts_hbm.at[row])

    return hist(ids3)


def _tc_body(counts_ref, emb_ref, wr1_ref, br1_ref, wp_ref, bp_ref,
             w1_ref, b1_ref, w2_ref, b2_ref,
             final_ref, logits_ref, act_ref,
             acc, facc, act_s):
    t = pl.program_id(0)

    @pl.when(t == 0)
    def _():
        acc[...] = jnp.zeros_like(acc)

    @pl.when(t < NPOOL)
    def _():
        c = counts_ref[...]
        e = emb_ref[...]
        def dd(a, b):
            return lax.dot_general(
                a, b, (((1,), (0,)), ((), ())),
                preferred_element_type=jnp.float32,
                precision=lax.Precision.DEFAULT)

        cB = jnp.floor(c * (1.0 / 256.0)) * 256.0
        cA = c - cB
        NCH = 2
        CH = VB // NCH
        part = jnp.zeros((B, EMBED), jnp.float32)
        for i in range(NCH):
            ei = e[i * CH:(i + 1) * CH, :]
            ci = cA[:, i * CH:(i + 1) * CH]
            e1 = ei.astype(jnp.bfloat16).astype(jnp.float32)
            r1 = ei - e1
            e2 = r1.astype(jnp.bfloat16).astype(jnp.float32)
            r2 = r1 - e2
            part += dd(ci, e1) + dd(ci, e2) + dd(ci, r2)
        acc[...] += part

        @pl.when(jnp.max(c) > 255.5)
        def _():
            e1 = e.astype(jnp.bfloat16).astype(jnp.float32)
            r1 = e - e1
            e2 = r1.astype(jnp.bfloat16).astype(jnp.float32)
            r2 = r1 - e2
            acc[...] += dd(cB, e1) + dd(cB, e2) + dd(cB, r2)

    @pl.when(t == NPOOL - 1)
    def _():
        x = acc[...] * (1.0 / S)
        acc[...] = x
        xb = x.astype(jnp.bfloat16)
        h = jnp.maximum(
            lax.dot_general(xb, wr1_ref[...].astype(jnp.bfloat16),
                            (((1,), (0,)), ((), ())),
                            preferred_element_type=jnp.float32)
            + br1_ref[...], 0.0)
        lg = lax.dot_general(
            h.astype(jnp.bfloat16), wp_ref[...].astype(jnp.bfloat16),
            (((1,), (0,)), ((), ())),
            preferred_element_type=jnp.float32) + bp_ref[...]
        logits_ref[...] = lg
        col = lax.broadcasted_iota(jnp.int32, lg.shape, 1)
        m = lg
        a = jnp.zeros_like(lg)
        for _ in range(TOP_K):
            mx = jnp.max(m, axis=1, keepdims=True)
            first = jnp.min(jnp.where(m == mx, col, NM), axis=1, keepdims=True)
            sel = col == first
            a = jnp.where(sel, 1.0, a)
            m = jnp.where(sel, -jnp.inf, m)
        act_ref[...] = a
        act_s[...] = a

    @pl.when(t >= NPOOL)
    def _():
        e_idx = t - NPOOL
        xb = acc[...].astype(jnp.bfloat16)
        h1 = jnp.maximum(
            lax.dot_general(xb, w1_ref[0].astype(jnp.bfloat16),
                            (((1,), (0,)), ((), ())),
                            preferred_element_type=jnp.float32)
            + b1_ref[0], 0.0)
        o = lax.dot_general(
            h1.astype(jnp.bfloat16), w2_ref[0].astype(jnp.bfloat16),
            (((1,), (0,)), ((), ())),
            preferred_element_type=jnp.float32) + b2_ref[0]
        col = lax.broadcasted_iota(jnp.int32, act_s.shape, 1)
        w = jnp.sum(jnp.where(col == e_idx, act_s[...], 0.0),
                    axis=1, keepdims=True)

        @pl.when(e_idx == 0)
        def _():
            facc[...] = w * o

        @pl.when(e_idx > 0)
        def _():
            facc[...] += w * o

        @pl.when(t == GRID - 1)
        def _():
            final_ref[...] = facc[...]


def kernel(input_ids, working_memory, affective_context, emb_table,
           Wr1, br1, Wp, bp, Wsg, W1, b1, W2, b2):
    counts = _histogram(input_ids)

    def _pool_i(t):
        return jnp.minimum(t, NPOOL - 1)

    def _mod_i(t):
        return jnp.clip(t - NPOOL, 0, NM - 1)

    final, logits, act = pl.pallas_call(
        _tc_body,
        grid=(GRID,),
        in_specs=[
            pl.BlockSpec((B, VB), lambda t: (0, _pool_i(t))),
            pl.BlockSpec((VB, EMBED), lambda t: (_pool_i(t), 0)),
            pl.BlockSpec((EMBED, RH), lambda t: (0, 0)),
            pl.BlockSpec((1, RH), lambda t: (0, 0)),
            pl.BlockSpec((RH, NM), lambda t: (0, 0)),
            pl.BlockSpec((1, NM), lambda t: (0, 0)),
            pl.BlockSpec((1, EMBED, EMBED), lambda t: (_mod_i(t), 0, 0)),
            pl.BlockSpec((1, 1, EMBED), lambda t: (_mod_i(t), 0, 0)),
            pl.BlockSpec((1, EMBED, EMBED), lambda t: (_mod_i(t), 0, 0)),
            pl.BlockSpec((1, 1, EMBED), lambda t: (_mod_i(t), 0, 0)),
        ],
        out_specs=[
            pl.BlockSpec((B, EMBED), lambda t: (0, 0)),
            pl.BlockSpec((B, NM), lambda t: (0, 0)),
            pl.BlockSpec((B, NM), lambda t: (0, 0)),
        ],
        out_shape=[
            jax.ShapeDtypeStruct((B, EMBED), jnp.float32),
            jax.ShapeDtypeStruct((B, NM), jnp.float32),
            jax.ShapeDtypeStruct((B, NM), jnp.float32),
        ],
        scratch_shapes=[
            pltpu.VMEM((B, EMBED), jnp.float32),
            pltpu.VMEM((B, EMBED), jnp.float32),
            pltpu.VMEM((B, NM), jnp.float32),
        ],
    )(counts, emb_table, Wr1, br1.reshape(1, RH), Wp, bp.reshape(1, NM),
      W1, b1.reshape(NM, 1, EMBED), W2, b2.reshape(NM, 1, EMBED))
    return final, logits, act


# speedup vs baseline: 1.1384x; 1.0002x over previous
import dataclasses
import functools

import jax
import jax.numpy as jnp
from jax import lax
from jax.experimental import pallas as pl
from jax.experimental.pallas import tpu as pltpu
from jax.experimental.pallas import tpu_sc as plsc

B = 32
S = 2048
VOCAB = 32000
EMBED = 1024
RH = 512
NM = 12
TOP_K = 3

VB = 1280
NPOOL = VOCAB // VB
NPOOL_A = 13
NPOOL_B = NPOOL - NPOOL_A
VSPLIT = NPOOL_A * VB
GRID2 = NPOOL_B + NM

_LANES = 16


def _histogram_half(ids3, lo, width):
    mesh = plsc.VectorSubcoreMesh(core_axis_name="c", subcore_axis_name="s")
    cp = pltpu.CompilerParams()
    if "needs_layout_passes" in pltpu.CompilerParams.__dataclass_fields__:
        cp = dataclasses.replace(cp, needs_layout_passes=False)

    @functools.partial(
        pl.kernel,
        out_type=jax.ShapeDtypeStruct((B, width), jnp.float32),
        mesh=mesh,
        compiler_params=cp,
        scratch_types=[
            pltpu.VMEM((S // _LANES, _LANES), jnp.int32),
            pltpu.VMEM((width,), jnp.float32),
            pltpu.SemaphoreType.DMA,
        ],
    )
    def hist(ids_hbm, counts_hbm, ids_v, counts_v, sem):
        cid = lax.axis_index("c")
        sid = lax.axis_index("s")
        row = cid * 16 + sid
        cp_ids = pltpu.async_copy(ids_hbm.at[row], ids_v, sem)

        zeros = jnp.zeros((_LANES,), jnp.float32)
        _ZU = 8

        @pl.loop(0, width // (_ZU * _LANES))
        def _(i):
            for u in range(_ZU):
                counts_v[pl.ds(i * (_ZU * _LANES) + u * _LANES, _LANES)] = zeros

        cp_ids.wait()
        ones = jnp.ones((_LANES,), jnp.float32)
        _SU = 4

        @pl.loop(0, S // (_SU * _LANES))
        def _(j):
            for u in range(_SU):
                v = ids_v[j * _SU + u, :] - lo
                mask = (v >= 0) & (v < width)
                idx = jnp.where(mask, v, 0)
                plsc.addupdate_scatter(counts_v, [idx], ones, mask=mask)

        pltpu.sync_copy(counts_v, counts_hbm.at[row])

    return hist(ids3)


def _dd(a, b):
    return lax.dot_general(
        a, b, (((1,), (0,)), ((), ())),
        preferred_element_type=jnp.float32,
        precision=lax.Precision.DEFAULT)


def _pool_block(c, e, acc):
    cB = jnp.floor(c * (1.0 / 256.0)) * 256.0
    cA = c - cB
    NCH = 2
    CH = VB // NCH
    part = jnp.zeros((B, EMBED), jnp.float32)
    for i in range(NCH):
        ei = e[i * CH:(i + 1) * CH, :]
        ci = cA[:, i * CH:(i + 1) * CH]
        e1 = ei.astype(jnp.bfloat16).astype(jnp.float32)
        r1 = ei - e1
        e2 = r1.astype(jnp.bfloat16).astype(jnp.float32)
        r2 = r1 - e2
        part += _dd(ci, e1) + _dd(ci, e2) + _dd(ci, r2)
    acc[...] += part

    @pl.when(jnp.max(c) > 255.5)
    def _():
        e1 = e.astype(jnp.bfloat16).astype(jnp.float32)
        r1 = e - e1
        e2 = r1.astype(jnp.bfloat16).astype(jnp.float32)
        r2 = r1 - e2
        acc[...] += _dd(cB, e1) + _dd(cB, e2) + _dd(cB, r2)


def _pool1_body(counts_ref, emb_ref, acc_ref, acc):
    t = pl.program_id(0)

    @pl.when(t == 0)
    def _():
        acc[...] = jnp.zeros_like(acc)

    _pool_block(counts_ref[...], emb_ref[...], acc)

    @pl.when(t == NPOOL_A - 1)
    def _():
        acc_ref[...] = acc[...]


def _tc2_body(counts_ref, emb_ref, acc1_ref, wr1_ref, br1_ref, wp_ref, bp_ref,
              w1_ref, b1_ref, w2_ref, b2_ref,
              final_ref, logits_ref, act_ref,
              acc, facc, act_s):
    t = pl.program_id(0)

    @pl.when(t == 0)
    def _():
        acc[...] = acc1_ref[...]

    @pl.when(t < NPOOL_B)
    def _():
        _pool_block(counts_ref[...], emb_ref[...], acc)

    @pl.when(t == NPOOL_B - 1)
    def _():
        x = acc[...] * (1.0 / S)
        acc[...] = x
        xb = x.astype(jnp.bfloat16)
        h = jnp.maximum(
            _dd(xb, wr1_ref[...].astype(jnp.bfloat16)) + br1_ref[...], 0.0)
        lg = _dd(h.astype(jnp.bfloat16),
                 wp_ref[...].astype(jnp.bfloat16)) + bp_ref[...]
        logits_ref[...] = lg
        col = lax.broadcasted_iota(jnp.int32, lg.shape, 1)
        m = lg
        a = jnp.zeros_like(lg)
        for _ in range(TOP_K):
            mx = jnp.max(m, axis=1, keepdims=True)
            first = jnp.min(jnp.where(m == mx, col, NM), axis=1, keepdims=True)
            sel = col == first
            a = jnp.where(sel, 1.0, a)
            m = jnp.where(sel, -jnp.inf, m)
        act_ref[...] = a
        act_s[...] = a

    @pl.when(t >= NPOOL_B)
    def _():
        e_idx = t - NPOOL_B
        xb = acc[...].astype(jnp.bfloat16)
        h1 = jnp.maximum(
            _dd(xb, w1_ref[0].astype(jnp.bfloat16)) + b1_ref[0], 0.0)
        o = _dd(h1.astype(jnp.bfloat16), w2_ref[0].astype(jnp.bfloat16)) \
            + b2_ref[0]
        col = lax.broadcasted_iota(jnp.int32, act_s.shape, 1)
        w = jnp.sum(jnp.where(col == e_idx, act_s[...], 0.0),
                    axis=1, keepdims=True)

        @pl.when(e_idx == 0)
        def _():
            facc[...] = w * o

        @pl.when(e_idx > 0)
        def _():
            facc[...] += w * o

        @pl.when(t == GRID2 - 1)
        def _():
            final_ref[...] = facc[...]


def kernel(input_ids, working_memory, affective_context, emb_table,
           Wr1, br1, Wp, bp, Wsg, W1, b1, W2, b2):
    ids3 = input_ids.reshape(B, S // _LANES, _LANES)
    counts_a = _histogram_half(ids3, 0, VSPLIT)
    counts_b = _histogram_half(ids3, VSPLIT, VOCAB - VSPLIT)

    acc1 = pl.pallas_call(
        _pool1_body,
        grid=(NPOOL_A,),
        in_specs=[
            pl.BlockSpec((B, VB), lambda t: (0, t)),
            pl.BlockSpec((VB, EMBED), lambda t: (t, 0)),
        ],
        out_specs=pl.BlockSpec((B, EMBED), lambda t: (0, 0)),
        out_shape=jax.ShapeDtypeStruct((B, EMBED), jnp.float32),
        scratch_shapes=[pltpu.VMEM((B, EMBED), jnp.float32)],
    )(counts_a, emb_table)

    def _pool_i(t):
        return jnp.minimum(t, NPOOL_B - 1)

    def _mod_i(t):
        return jnp.clip(t - NPOOL_B, 0, NM - 1)

    final, logits, act = pl.pallas_call(
        _tc2_body,
        grid=(GRID2,),
        in_specs=[
            pl.BlockSpec((B, VB), lambda t: (0, _pool_i(t))),
            pl.BlockSpec((VB, EMBED), lambda t: (NPOOL_A + _pool_i(t), 0)),
            pl.BlockSpec((B, EMBED), lambda t: (0, 0)),
            pl.BlockSpec((EMBED, RH), lambda t: (0, 0)),
            pl.BlockSpec((1, RH), lambda t: (0, 0)),
            pl.BlockSpec((RH, NM), lambda t: (0, 0)),
            pl.BlockSpec((1, NM), lambda t: (0, 0)),
            pl.BlockSpec((1, EMBED, EMBED), lambda t: (_mod_i(t), 0, 0)),
            pl.BlockSpec((1, 1, EMBED), lambda t: (_mod_i(t), 0, 0)),
            pl.BlockSpec((1, EMBED, EMBED), lambda t: (_mod_i(t), 0, 0)),
            pl.BlockSpec((1, 1, EMBED), lambda t: (_mod_i(t), 0, 0)),
        ],
        out_specs=[
            pl.BlockSpec((B, EMBED), lambda t: (0, 0)),
            pl.BlockSpec((B, NM), lambda t: (0, 0)),
            pl.BlockSpec((B, NM), lambda t: (0, 0)),
        ],
        out_shape=[
            jax.ShapeDtypeStruct((B, EMBED), jnp.float32),
            jax.ShapeDtypeStruct((B, NM), jnp.float32),
            jax.ShapeDtypeStruct((B, NM), jnp.float32),
        ],
        scratch_shapes=[
            pltpu.VMEM((B, EMBED), jnp.float32),
            pltpu.VMEM((B, EMBED), jnp.float32),
            pltpu.VMEM((B, NM), jnp.float32),
        ],
    )(counts_b, emb_table, acc1, Wr1, br1.reshape(1, RH), Wp,
      bp.reshape(1, NM), W1, b1.reshape(NM, 1, EMBED), W2,
      b2.reshape(NM, 1, EMBED))
    return final, logits, act
